# baseline (speedup 1.0000x reference)
import functools

import jax
import jax.numpy as jnp
from jax import lax
from jax.experimental import pallas as pl
from jax.experimental.pallas import tpu as pltpu
from jax.experimental.pallas import tpu_sc as plsc

_CHUNK = 128
_LANES = 16


def _vsplat(v, lane):
    return lax.gather(
        v, jnp.full((_LANES, 1), lane, jnp.int32),
        lax.GatherDimensionNumbers(
            offset_dims=(), collapsed_slice_dims=(0,), start_index_map=(0,)),
        slice_sizes=(1,), mode=lax.GatherScatterMode.PROMISE_IN_BOUNDS)


def _sc_motor_kernel(n_motor, n_conn, batch, n_neurons, b_lo):
    info = plsc.get_sparse_core_info()
    nc, ns = info.num_cores, info.num_subcores
    nw = nc * ns
    npw = n_motor // nw
    rpw = npw * n_conn
    epw = rpw * batch
    n_chunks = epw // _CHUNK
    ppc = _CHUNK // batch
    cpr = n_conn // ppc
    assert n_motor % nw == 0 and epw % _CHUNK == 0 and batch == _LANES
    assert n_conn % _LANES == 0 and npw <= _LANES

    mesh = plsc.VectorSubcoreMesh(core_axis_name="c", subcore_axis_name="s")

    @functools.partial(
        pl.kernel,
        out_type=jax.ShapeDtypeStruct((n_motor, batch), jnp.float32),
        mesh=mesh,
        compiler_params=pltpu.CompilerParams(use_tc_tiling_on_sc=False),
        scratch_types=[
            pltpu.VMEM((npw, n_conn), jnp.int32),
            pltpu.VMEM((npw, n_conn), jnp.float32),
            pltpu.VMEM((_LANES,), jnp.float32),
            pltpu.VMEM((n_chunks, _CHUNK), jnp.int32),
            pltpu.VMEM((epw,), jnp.float32),
            pltpu.VMEM((npw, batch), jnp.float32),
            pltpu.SemaphoreType.DMA,
            pltpu.SemaphoreType.DMA,
        ],
    )
    def k(idx_hbm, w_hbm, b_hbm, xf_hbm, out_hbm,
          idx_l, w_l, b_l, idx_v, elems_v, out_v, sem, sem2):
        wid = lax.axis_index("s") * nc + lax.axis_index("c")
        pltpu.sync_copy(idx_hbm.at[pl.ds(wid * npw, npw)], idx_l)
        cp_w = pltpu.async_copy(w_hbm.at[pl.ds(wid * npw, npw)], w_l, sem2)
        cp_b = pltpu.async_copy(
            b_hbm.at[pl.ds(b_lo + wid * npw, npw)],
            b_l.at[pl.ds(0, npw)], sem2)

        offs = lax.iota(jnp.int32, _LANES) * n_neurons
        for c in range(n_chunks):
            n = c // cpr
            col = (c % cpr) * ppc
            iv = idx_l[n, pl.ds((col // _LANES) * _LANES, _LANES)]
            for j in range(ppc):
                flat = _vsplat(iv, col % _LANES + j) + offs
                idx_v[c, pl.ds(j * batch, batch)] = flat
            pltpu.async_copy(
                xf_hbm.at[idx_v.at[c]],
                elems_v.at[pl.ds(c * _CHUNK, _CHUNK)], sem)

        cp_w.wait()
        cp_b.wait()
        pltpu.make_async_copy(xf_hbm.at[pl.ds(0, epw)], elems_v, sem).wait()

        bv = b_l[...]
        for n in range(npw):
            acc = _vsplat(bv, n)
            for h in range(n_conn // _LANES):
                wv = w_l[n, pl.ds(h * _LANES, _LANES)]
                for j in range(_LANES):
                    p = n * n_conn + h * _LANES + j
                    acc = acc + (elems_v[pl.ds(p * batch, batch)]
                                 * _vsplat(wv, j))
            out_v[n, :] = 1.0 / (1.0 + jnp.exp(-acc))
        pltpu.sync_copy(out_v, out_hbm.at[pl.ds(wid * npw, npw)])

    return k


def _make_flatten(batch, n_neurons, stride):
    main = n_neurons // 128 * 128

    def _flatten(x_ref, t_ref, o_ref, sem):
        cps = []
        for b in range(batch):
            cps.append(pltpu.make_async_copy(
                x_ref.at[b, pl.ds(0, main)],
                o_ref.at[pl.ds(b * stride, main)], sem))
            if main < n_neurons:
                cps.append(pltpu.make_async_copy(
                    t_ref.at[b, :],
                    o_ref.at[pl.ds(b * stride + main, 128)], sem))
        for cp in cps:
            cp.start()
        for cp in cps:
            cp.wait()
    return _flatten


def _q_head(m_ref, wq_ref, bq_ref, o_ref):
    q = lax.dot_general(
        m_ref[...], wq_ref[...],
        dimension_numbers=(((0,), (1,)), ((), ())),
        preferred_element_type=jnp.float32,
    )
    o_ref[...] = q + bq_ref[...][None, :]


def kernel(x, W_think, b_think, idx, W_q, b_q):
    batch, n_neurons = x.shape
    n_actions, n_motor = W_q.shape
    out_f, n_conn = idx.shape

    lo = out_f - n_motor
    idx_m = idx[lo:]
    w_m = W_think[lo:]
    main = n_neurons // 128 * 128
    stride = main + 128
    assert stride >= n_neurons
    xtails = jnp.pad(x[:, main:], ((0, 0), (0, 128 - (n_neurons - main))))
    xf = pl.pallas_call(
        _make_flatten(batch, n_neurons, stride),
        in_specs=[pl.BlockSpec(memory_space=pl.ANY),
                  pl.BlockSpec(memory_space=pl.ANY)],
        out_specs=pl.BlockSpec(memory_space=pl.ANY),
        out_shape=jax.ShapeDtypeStruct((batch * stride,), jnp.float32),
        scratch_shapes=[pltpu.SemaphoreType.DMA],
    )(x, xtails)

    motor_f = _sc_motor_kernel(n_motor, n_conn, batch, stride, lo)(
        idx_m, w_m, b_think, xf)

    q = pl.pallas_call(
        _q_head,
        out_shape=jax.ShapeDtypeStruct((batch, n_actions), jnp.float32),
    )(motor_f, W_q, b_q)
    return q


# speedup vs baseline: 5.7872x; 5.7872x over previous
import functools

import jax
import jax.numpy as jnp
from jax import lax
from jax.experimental import pallas as pl
from jax.experimental.pallas import tpu as pltpu
from jax.experimental.pallas import tpu_sc as plsc

_CHUNK = 128
_LANES = 16


def _vsplat(v, lane):
    return lax.gather(
        v, jnp.full((_LANES, 1), lane, jnp.int32),
        lax.GatherDimensionNumbers(
            offset_dims=(), collapsed_slice_dims=(0,), start_index_map=(0,)),
        slice_sizes=(1,), mode=lax.GatherScatterMode.PROMISE_IN_BOUNDS)


def _sc_motor_kernel(n_motor, n_conn, batch, n_neurons, b_lo):
    info = plsc.get_sparse_core_info()
    nc, ns = info.num_cores, info.num_subcores
    nw = nc * ns
    npw = n_motor // nw
    rpw = npw * n_conn
    epw = rpw * batch
    n_chunks = epw // _CHUNK
    ppc = _CHUNK // batch
    cpr = n_conn // ppc
    assert n_motor % nw == 0 and epw % _CHUNK == 0 and batch == _LANES
    assert n_conn % _LANES == 0 and npw <= _LANES

    mesh = plsc.VectorSubcoreMesh(core_axis_name="c", subcore_axis_name="s")

    @functools.partial(
        pl.kernel,
        out_type=jax.ShapeDtypeStruct((n_motor, batch), jnp.float32),
        mesh=mesh,
        compiler_params=pltpu.CompilerParams(use_tc_tiling_on_sc=False),
        scratch_types=[
            pltpu.VMEM((npw, n_conn), jnp.int32),
            pltpu.VMEM((npw, n_conn), jnp.float32),
            pltpu.VMEM((_LANES,), jnp.float32),
            pltpu.VMEM((epw,), jnp.int32),
            pltpu.VMEM((epw,), jnp.float32),
            pltpu.VMEM((npw, batch), jnp.float32),
            pltpu.SemaphoreType.DMA,
            pltpu.SemaphoreType.DMA,
        ],
    )
    def k(idx_hbm, w_hbm, b_hbm, xf_hbm, out_hbm,
          idx_l, w_l, b_l, idx_v, elems_v, out_v, sem, sem2):
        wid = lax.axis_index("s") * nc + lax.axis_index("c")
        pltpu.sync_copy(idx_hbm.at[pl.ds(wid * npw, npw)], idx_l)
        cp_w = pltpu.async_copy(w_hbm.at[pl.ds(wid * npw, npw)], w_l, sem2)
        cp_b = pltpu.async_copy(
            b_hbm.at[pl.ds(b_lo + wid * npw, npw)],
            b_l.at[pl.ds(0, npw)], sem2)

        qsz = epw // 4
        offs = lax.iota(jnp.int32, _LANES) * n_neurons
        for p in range(rpw):
            iv = idx_l[p // n_conn,
                       pl.ds((p % n_conn) // _LANES * _LANES, _LANES)]
            flat = _vsplat(iv, p % _LANES) + offs
            idx_v[pl.ds(p * batch, batch)] = flat
            if (p + 1) * batch % qsz == 0:
                q = (p + 1) * batch // qsz - 1
                pltpu.async_copy(
                    xf_hbm.at[idx_v.at[pl.ds(q * qsz, qsz)]],
                    elems_v.at[pl.ds(q * qsz, qsz)], sem)

        cp_w.wait()
        cp_b.wait()
        pltpu.make_async_copy(xf_hbm.at[pl.ds(0, epw)], elems_v, sem).wait()

        bv = b_l[...]
        for n in range(npw):
            acc = _vsplat(bv, n)
            for h in range(n_conn // _LANES):
                wv = w_l[n, pl.ds(h * _LANES, _LANES)]
                for j in range(_LANES):
                    p = n * n_conn + h * _LANES + j
                    acc = acc + (elems_v[pl.ds(p * batch, batch)]
                                 * _vsplat(wv, j))
            out_v[n, :] = 1.0 / (1.0 + jnp.exp(-acc))
        pltpu.sync_copy(out_v, out_hbm.at[pl.ds(wid * npw, npw)])

    return k


def _q_head(m_ref, wq_ref, bq_ref, o_ref):
    q = lax.dot_general(
        m_ref[...], wq_ref[...],
        dimension_numbers=(((0,), (1,)), ((), ())),
        preferred_element_type=jnp.float32,
    )
    o_ref[...] = q + bq_ref[...][None, :]


def kernel(x, W_think, b_think, idx, W_q, b_q):
    batch, n_neurons = x.shape
    n_actions, n_motor = W_q.shape
    out_f, n_conn = idx.shape

    lo = out_f - n_motor
    idx_m = idx[lo:]
    w_m = W_think[lo:]
    xf = x.reshape(-1)

    motor_f = _sc_motor_kernel(n_motor, n_conn, batch, n_neurons, lo)(
        idx_m, w_m, b_think, xf)

    q = pl.pallas_call(
        _q_head,
        out_shape=jax.ShapeDtypeStruct((batch, n_actions), jnp.float32),
    )(motor_f, W_q, b_q)
    return q
